# baseline (speedup 1.0000x reference)
import jax
import jax.numpy as jnp
from jax import lax
from jax.experimental import pallas as pl
from jax.experimental.pallas import tpu as pltpu
from jax.experimental.pallas import tpu_sc as plsc

EMBED = 64
SEQ = 200
RW = 100
NW = 32
CR = 4
CHUNK = CR * RW
NB = 3
LANES = 16


def _body(x_hbm, tok_hbm, pos_hbm, out_hbm, idx_v, pos_v, rows_v,
          sem0, sem1, sem2):
    sems = (sem0, sem1, sem2)
    rpw = x_hbm.shape[0] // NW
    nch = rpw // CR
    wid = lax.axis_index("s") * 2 + lax.axis_index("c")
    base_row = wid * rpw

    pltpu.sync_copy(x_hbm.at[pl.ds(base_row, rpw)], idx_v)
    pltpu.sync_copy(pos_hbm, pos_v)

    def fire(g, b):
        for j in range(CR):
            pltpu.async_copy(
                tok_hbm.at[idx_v.at[g * CR + j]],
                rows_v.at[pl.ds(b * CHUNK + j * RW, RW)],
                sems[b])

    def drain(g, b):
        for j in range(CR):
            pltpu.make_async_copy(
                tok_hbm.at[idx_v.at[g * CR + j]],
                rows_v.at[pl.ds(b * CHUNK + j * RW, RW)],
                sems[b]).wait()

    def add_pos(b):
        def body(r, carry):
            for h in range(CHUNK // SEQ):
                row = b * CHUNK + h * SEQ + r
                for c in range(EMBED // LANES):
                    sl = pl.ds(c * LANES, LANES)
                    rows_v[row, sl] = rows_v[row, sl] + pos_v[r, sl]
            return carry
        lax.fori_loop(0, SEQ, body, 0)

    for b in range(NB):
        fire(b, b)

    def outer(i, carry):
        for b in range(NB):
            g = i * NB + b

            @pl.when(g < nch)
            def _process():
                drain(g, b)
                add_pos(b)
                pltpu.sync_copy(
                    rows_v.at[pl.ds(b * CHUNK, CHUNK)],
                    out_hbm.at[pl.ds((base_row + g * CR) * RW, CHUNK)])

                @pl.when(g + NB < nch)
                def _fire_ahead():
                    fire(g + NB, b)
        return carry

    lax.fori_loop(0, (nch + NB - 1) // NB, outer, 0)


def _impl(x2d, tok, pos):
    rows = x2d.shape[0]
    rpw = rows // NW
    mesh = plsc.VectorSubcoreMesh(core_axis_name="c", subcore_axis_name="s")
    f = pl.kernel(
        _body,
        mesh=mesh,
        out_type=jax.ShapeDtypeStruct((rows * RW, EMBED), jnp.float32),
        scratch_types=[
            pltpu.VMEM((rpw, RW), jnp.int32),
            pltpu.VMEM((SEQ, EMBED), jnp.float32),
            pltpu.VMEM((NB * CHUNK, EMBED), jnp.float32),
            pltpu.SemaphoreType.DMA,
            pltpu.SemaphoreType.DMA,
            pltpu.SemaphoreType.DMA,
        ],
        compiler_params=pltpu.CompilerParams(use_tc_tiling_on_sc=False),
    )
    return f(x2d, tok, pos)


def kernel(x, token_table, pos_table):
    batch, seq = x.shape
    x2d = x.reshape(batch * seq // RW, RW).astype(jnp.int32)
    out = _impl(x2d, token_table, pos_table)
    return out.reshape(batch, seq, EMBED)
